# baseline (speedup 1.0000x reference)
import functools

import jax
import jax.numpy as jnp
from jax import lax
from jax.experimental import pallas as pl
from jax.experimental.pallas import tpu as pltpu

MAXK_ = 8
MINK_ = 1
_P = lax.Precision.DEFAULT


def _dot(a, b):
    return jnp.dot(a, b, preferred_element_type=jnp.float32, precision=_P)


def _body(x_ref, rwt_ref, wc_ref, bc_ref, w2_ref, b2_ref,
          w3_ref, b3_ref, w4c_ref, w5_ref, b5_ref, out_w_ref, out_i_ref):
    x = x_ref[...].astype(jnp.bfloat16)
    D2 = w2_ref.shape[0]
    wcb = wc_ref[...].astype(jnp.bfloat16)
    xc = _dot(x, wcb) + bc_ref[...]
    h1 = jnp.maximum(xc[:, :D2], 0.0)
    gpre = xc[:, D2:]
    h2 = jnp.maximum(
        _dot(h1.astype(jnp.bfloat16), w2_ref[...].astype(jnp.bfloat16))
        + b2_ref[...], 0.0)
    c = jax.nn.sigmoid(_dot(h2, w3_ref[...]) + b3_ref[...])
    g = jnp.maximum(gpre + _dot(c, w4c_ref[...]), 0.0)
    z5t = lax.dot_general(w5_ref[...], g, (((1,), (1,)), ((), ())),
                          precision=_P,
                          preferred_element_type=jnp.float32)
    r = jax.nn.sigmoid(z5t + b5_ref[...])
    counts = jnp.round(MINK_ + r * (MAXK_ - MINK_))

    cur = rwt_ref[...]
    E, T = cur.shape
    iota = lax.broadcasted_iota(jnp.int32, (E, T), 0).astype(jnp.float32)
    j8 = lax.broadcasted_iota(jnp.int32, (MAXK_, T), 0).astype(jnp.float32)
    top_w = jnp.zeros((MAXK_, T), jnp.float32)
    top_i = jnp.zeros((MAXK_, T), jnp.float32)
    for j in range(MAXK_):
        m = jnp.max(cur, axis=0, keepdims=True)
        eq = cur == m
        idx = jnp.min(jnp.where(eq, iota, float(E)), axis=0, keepdims=True)
        top_w = jnp.where(j8 == j, m, top_w)
        top_i = jnp.where(j8 == j, idx, top_i)
        if j + 1 < MAXK_:
            cur = jnp.where(iota == idx, -jnp.inf, cur)

    mask = (j8 < counts).astype(jnp.float32)
    masked = top_w * mask
    s = jnp.sum(masked, axis=0, keepdims=True)
    s = jnp.where(s > 0.0, s, 1.0)
    out_w_ref[...] = masked / s
    out_i_ref[...] = top_i.astype(jnp.int32)


@functools.partial(jax.jit, static_argnames=("interpret",))
def kernel(x, routing_weights, W1, b1, W2, b2, W3, b3, W4, b4, W5, b5,
           interpret=False):
    B, S, D = x.shape
    E = routing_weights.shape[-1]
    N = B * S
    D2, D4 = W1.shape[1], W2.shape[1]
    T = 2048

    xf = x.reshape(N, D)
    rwt = routing_weights.reshape(N, E).T
    wc = jnp.concatenate([W1, W4[:D]], axis=1)
    bc = jnp.concatenate([b1, b4]).reshape(1, 2 * D2)
    w4c = W4[D].reshape(1, D2)
    w5 = W5.reshape(1, D2)

    grid = (N // T,)
    full = lambda shape: pl.BlockSpec(shape, lambda i: tuple(0 for _ in shape))
    out_w, out_i = pl.pallas_call(
        _body,
        grid=grid,
        in_specs=[
            pl.BlockSpec((T, D), lambda i: (i, 0)),
            pl.BlockSpec((E, T), lambda i: (0, i)),
            full((D, 2 * D2)),
            full((1, 2 * D2)),
            full((D2, D4)),
            full((1, D4)),
            full((D4, 1)),
            full((1, 1)),
            full((1, D2)),
            full((1, D2)),
            full((1, 1)),
        ],
        out_specs=[
            pl.BlockSpec((MAXK_, T), lambda i: (0, i)),
            pl.BlockSpec((MAXK_, T), lambda i: (0, i)),
        ],
        out_shape=[
            jax.ShapeDtypeStruct((MAXK_, N), jnp.float32),
            jax.ShapeDtypeStruct((MAXK_, N), jnp.int32),
        ],
        compiler_params=pltpu.CompilerParams(
            dimension_semantics=("arbitrary",),
        ),
        interpret=interpret,
    )(xf, rwt, wc, bc, W2, b2.reshape(1, D4), W3, b3.reshape(1, 1),
      w4c, w5, b5.reshape(1, 1))
    return (out_w.T.reshape(B, S, MAXK_), out_i.T.reshape(B, S, MAXK_))


# speedup vs baseline: 1.0056x; 1.0056x over previous
import functools

import jax
import jax.numpy as jnp
from jax import lax
from jax.experimental import pallas as pl
from jax.experimental.pallas import tpu as pltpu

MAXK_ = 8
MINK_ = 1
_P = lax.Precision.DEFAULT


def _dot(a, b):
    return jnp.dot(a, b, preferred_element_type=jnp.float32, precision=_P)


def _body(x_ref, rwt_ref, wc_ref, bc_ref, w2_ref, b2_ref,
          w3_ref, b3_ref, w4c_ref, w5_ref, b5_ref, out_w_ref, out_i_ref):
    x = x_ref[...].astype(jnp.bfloat16)
    D2 = w2_ref.shape[0]
    wcb = wc_ref[...].astype(jnp.bfloat16)
    xc = _dot(x, wcb) + bc_ref[...]
    h1 = jnp.maximum(xc[:, :D2], 0.0)
    gpre = xc[:, D2:]
    h2 = jnp.maximum(
        _dot(h1.astype(jnp.bfloat16), w2_ref[...].astype(jnp.bfloat16))
        + b2_ref[...], 0.0)
    c = jax.nn.sigmoid(_dot(h2, w3_ref[...]) + b3_ref[...])
    g = jnp.maximum(gpre + _dot(c, w4c_ref[...]), 0.0)
    z5t = lax.dot_general(w5_ref[...], g, (((1,), (1,)), ((), ())),
                          precision=_P,
                          preferred_element_type=jnp.float32)
    r = jax.nn.sigmoid(z5t + b5_ref[...])
    counts = jnp.round(MINK_ + r * (MAXK_ - MINK_))

    cur = jnp.transpose(rwt_ref[...])
    E, T = cur.shape
    iota = lax.broadcasted_iota(jnp.int32, (E, T), 0).astype(jnp.float32)
    j8 = lax.broadcasted_iota(jnp.int32, (MAXK_, T), 0).astype(jnp.float32)
    top_w = jnp.zeros((MAXK_, T), jnp.float32)
    top_i = jnp.zeros((MAXK_, T), jnp.float32)
    for j in range(MAXK_):
        m = jnp.max(cur, axis=0, keepdims=True)
        eq = cur == m
        idx = jnp.min(jnp.where(eq, iota, float(E)), axis=0, keepdims=True)
        top_w = jnp.where(j8 == j, m, top_w)
        top_i = jnp.where(j8 == j, idx, top_i)
        if j + 1 < MAXK_:
            cur = jnp.where(iota == idx, -jnp.inf, cur)

    mask = (j8 < counts).astype(jnp.float32)
    masked = top_w * mask
    s = jnp.sum(masked, axis=0, keepdims=True)
    s = jnp.where(s > 0.0, s, 1.0)
    out_w_ref[...] = masked / s
    out_i_ref[...] = top_i.astype(jnp.int32)


@functools.partial(jax.jit, static_argnames=("interpret",))
def kernel(x, routing_weights, W1, b1, W2, b2, W3, b3, W4, b4, W5, b5,
           interpret=False):
    B, S, D = x.shape
    E = routing_weights.shape[-1]
    N = B * S
    D2, D4 = W1.shape[1], W2.shape[1]
    T = 2048

    xf = x.reshape(N, D)
    rwt = routing_weights.reshape(N, E)
    wc = jnp.concatenate([W1, W4[:D]], axis=1)
    bc = jnp.concatenate([b1, b4]).reshape(1, 2 * D2)
    w4c = W4[D].reshape(1, D2)
    w5 = W5.reshape(1, D2)

    grid = (N // T,)
    full = lambda shape: pl.BlockSpec(shape, lambda i: tuple(0 for _ in shape))
    out_w, out_i = pl.pallas_call(
        _body,
        grid=grid,
        in_specs=[
            pl.BlockSpec((T, D), lambda i: (i, 0)),
            pl.BlockSpec((T, E), lambda i: (i, 0)),
            full((D, 2 * D2)),
            full((1, 2 * D2)),
            full((D2, D4)),
            full((1, D4)),
            full((D4, 1)),
            full((1, 1)),
            full((1, D2)),
            full((1, D2)),
            full((1, 1)),
        ],
        out_specs=[
            pl.BlockSpec((MAXK_, T), lambda i: (0, i)),
            pl.BlockSpec((MAXK_, T), lambda i: (0, i)),
        ],
        out_shape=[
            jax.ShapeDtypeStruct((MAXK_, N), jnp.float32),
            jax.ShapeDtypeStruct((MAXK_, N), jnp.int32),
        ],
        compiler_params=pltpu.CompilerParams(
            dimension_semantics=("arbitrary",),
        ),
        interpret=interpret,
    )(xf, rwt, wc, bc, W2, b2.reshape(1, D4), W3, b3.reshape(1, 1),
      w4c, w5, b5.reshape(1, 1))
    return (out_w.T.reshape(B, S, MAXK_), out_i.T.reshape(B, S, MAXK_))
